# baseline (speedup 1.0000x reference)
import functools

import jax
import jax.numpy as jnp
from jax import lax
from jax.experimental import pallas as pl
from jax.experimental.pallas import tpu as pltpu
from jax.experimental.pallas import tpu_sc as plsc

N = 10000
D = 128
NP = 10240
TR = NP // 16
E = 320000
C = 64
GRP = 16
NW = 32
CW = ((-(-E // (NW * C)) + GRP - 1) // GRP) * GRP
NG = CW // GRP
EP = NW * CW * C
EROWS = EP // C
EC = 1024
NC2 = (EP // 2) // EC
HC = 64
NP2 = 10112
BLK = 256


def _proj_body(nf, wn, gh, wg, asrc, adst, ht_out, ss_out, sd_out, m_out,
               macc):
    i = pl.program_id(0)
    hg = gh[...] @ wg[...]
    hb = nf[...] @ wn[...] + hg
    ht_out[...] = jnp.transpose(hb)
    sb = hb @ asrc[...]
    db = hb @ adst[...]
    ss_out[...] = sb
    sd_out[...] = db

    @pl.when(i == 0)
    def _():
        macc[0] = jnp.float32(-3.4e38)
        macc[1] = jnp.float32(-3.4e38)

    macc[0] = jnp.maximum(macc[0], jnp.max(sb))
    macc[1] = jnp.maximum(macc[1], jnp.max(db))

    @pl.when(i == pl.num_programs(0) - 1)
    def _():
        m_out[...] = jnp.full((1, 1), jnp.maximum(macc[0] + macc[1], 0.0))


def _project(nf_pad, Wn, gh, Wg, asrc, adst):
    grid = NP // BLK
    return pl.pallas_call(
        _proj_body,
        grid=(grid,),
        in_specs=[
            pl.BlockSpec((BLK, D), lambda i: (i, 0)),
            pl.BlockSpec((D, D), lambda i: (0, 0)),
            pl.BlockSpec((1, D), lambda i: (0, 0)),
            pl.BlockSpec((D, D), lambda i: (0, 0)),
            pl.BlockSpec((D, 1), lambda i: (0, 0)),
            pl.BlockSpec((D, 1), lambda i: (0, 0)),
        ],
        out_specs=[
            pl.BlockSpec((D, BLK), lambda i: (0, i)),
            pl.BlockSpec((BLK, 1), lambda i: (i, 0)),
            pl.BlockSpec((BLK, 1), lambda i: (i, 0)),
            pl.BlockSpec((1, 1), lambda i: (0, 0)),
        ],
        out_shape=[
            jax.ShapeDtypeStruct((D, NP), jnp.float32),
            jax.ShapeDtypeStruct((NP, 1), jnp.float32),
            jax.ShapeDtypeStruct((NP, 1), jnp.float32),
            jax.ShapeDtypeStruct((1, 1), jnp.float32),
        ],
        scratch_shapes=[pltpu.SMEM((2,), jnp.float32)],
    )(nf_pad, Wn, gh, Wg, asrc, adst)


def _p1_body(srcm_hbm, dstm_hbm, ssrc_hbm, sdst_hbm, mval_hbm,
             w_out, den_out,
             ssrc_v, sdst_v, sg_src, sg_dst, wb, mv_v, zden, den_sh):
    cid = lax.axis_index("c")
    sid = lax.axis_index("s")
    wid = sid * 2 + cid

    pltpu.sync_copy(ssrc_hbm, ssrc_v)
    pltpu.sync_copy(sdst_hbm, sdst_v)
    pltpu.sync_copy(mval_hbm, mv_v)

    def _zden(r, _):
        zden[pl.ds(r * 16, 16)] = jnp.zeros((16,), jnp.float32)
        return 0
    lax.fori_loop(0, TR // 16, _zden, 0)
    pltpu.sync_copy(zden, den_sh.at[pl.ds(sid * TR, TR)])
    plsc.subcore_barrier()

    mvec = mv_v[...]

    def _group(g, _):
        base = wid * CW + g * GRP
        pltpu.sync_copy(srcm_hbm.at[pl.ds(base, GRP)], sg_src)
        pltpu.sync_copy(dstm_hbm.at[pl.ds(base, GRP)], sg_dst)

        def _chunk(j, _):
            for k in range(C // 16):
                si = sg_src[j, pl.ds(k * 16, 16)]
                di = sg_dst[j, pl.ds(k * 16, 16)]
                a = plsc.load_gather(ssrc_v, [si])
                b = plsc.load_gather(sdst_v, [di])
                pre = a + b
                e = jnp.where(pre >= 0.0, pre, 0.2 * pre)
                wb[pl.ds(k * 16, 16)] = jnp.exp(e - mvec)
            pltpu.sync_copy(wb, w_out.at[pl.ds((base + j) * C, C)])
            pltpu.sync_copy(wb, den_sh.at[sg_dst.at[j]], add=True)
            return 0

        lax.fori_loop(0, GRP, _chunk, 0)
        return 0

    lax.fori_loop(0, NG, _group, 0)
    plsc.subcore_barrier()
    pltpu.sync_copy(den_sh.at[pl.ds(sid * TR, TR)],
                    den_out.at[cid, pl.ds(sid * TR, TR)])


def _phase1(srcm, dstm, ssrc, sdst, mval):
    mesh = plsc.VectorSubcoreMesh(
        core_axis_name="c", subcore_axis_name="s", num_cores=2,
        num_subcores=16)
    f = pl.kernel(
        _p1_body,
        out_type=[
            jax.ShapeDtypeStruct((EP,), jnp.float32),
            jax.ShapeDtypeStruct((2, NP), jnp.float32),
        ],
        mesh=mesh,
        compiler_params=pltpu.CompilerParams(needs_layout_passes=False),
        scratch_types=[
            pltpu.VMEM((NP,), jnp.float32),
            pltpu.VMEM((NP,), jnp.float32),
            pltpu.VMEM((GRP, C), jnp.int32),
            pltpu.VMEM((GRP, C), jnp.int32),
            pltpu.VMEM((C,), jnp.float32),
            pltpu.VMEM((16,), jnp.float32),
            pltpu.VMEM((TR,), jnp.float32),
            pltpu.VMEM_SHARED((NP,), jnp.float32),
        ],
    )
    return f(srcm, dstm, ssrc, sdst, mval)


def _p2_body(ht_hbm, srcm_hbm, dstm_hbm, w_hbm, u_out,
             table, ubuf, sb0, db0, wb0, sb1, db1, wb1,
             sem0, sem1):
    cid = lax.axis_index("c")
    sid = lax.axis_index("s")

    pltpu.sync_copy(ht_hbm.at[pl.ds(sid * 8, 8), pl.ds(0, NP2)], ubuf)

    def _packcols(i, _):
        for j in range(4):
            a = ubuf[2 * j, pl.ds(i * 16, 16)]
            b = ubuf[2 * j + 1, pl.ds(i * 16, 16)]
            pk = plsc.pack(a, b, format=plsc.PackFormat.INTERLEAVED)
            table[j, pl.ds(i * 16, 16)] = plsc.bitcast(pk, jnp.int32)
        return 0
    lax.fori_loop(0, NP2 // 16, _packcols, 0)

    def _zu(r, _):
        for j in range(8):
            ubuf[j, pl.ds(r * 16, 16)] = jnp.zeros((16,), jnp.float32)
        return 0
    lax.fori_loop(0, NP2 // 16, _zu, 0)

    jconst = [jnp.full((16,), j, jnp.int32) for j in range(8)]
    erow0 = cid * (EP // 256)

    def _fire(ch, sb, db, wb, sem):
        rbase = erow0 + ch * (EC // 128)
        ebase = rbase * 128
        pltpu.async_copy(srcm_hbm.at[pl.ds(rbase, EC // 128)], sb, sem)
        pltpu.async_copy(dstm_hbm.at[pl.ds(rbase, EC // 128)], db, sem)
        pltpu.async_copy(w_hbm.at[pl.ds(ebase, EC)], wb, sem)

    def _drain(ch, sb, db, wb, sem):
        rbase = erow0 + ch * (EC // 128)
        ebase = rbase * 128
        pltpu.make_async_copy(
            srcm_hbm.at[pl.ds(rbase, EC // 128)], sb, sem).wait()
        pltpu.make_async_copy(
            dstm_hbm.at[pl.ds(rbase, EC // 128)], db, sem).wait()
        pltpu.make_async_copy(
            w_hbm.at[pl.ds(ebase, EC)], wb, sem).wait()

    def _process(sb, db, wb):
        def _row(r, _):
            for q in range(8):
                srcv = sb[r, pl.ds(q * 16, 16)]
                dstv = db[r, pl.ds(q * 16, 16)]
                wv = wb[pl.ds(r * 128 + q * 16, 16)]
                for j in range(4):
                    g = plsc.load_gather(table, [jconst[j], srcv])
                    bf = plsc.bitcast(g, jnp.bfloat16)
                    fa, fb = plsc.unpack(
                        bf, format=plsc.PackFormat.INTERLEAVED)
                    plsc.addupdate_scatter(
                        ubuf, [jconst[2 * j], dstv], wv * fa)
                    plsc.addupdate_scatter(
                        ubuf, [jconst[2 * j + 1], dstv], wv * fb)
            return 0
        lax.fori_loop(0, EC // 128, _row, 0)

    _fire(0, sb0, db0, wb0, sem0)

    def _loop(ch, _):
        @pl.when(jnp.logical_and(ch + 1 < NC2, ch % 2 == 0))
        def _():
            _fire(ch + 1, sb1, db1, wb1, sem1)

        @pl.when(jnp.logical_and(ch + 1 < NC2, ch % 2 == 1))
        def _():
            _fire(ch + 1, sb0, db0, wb0, sem0)

        @pl.when(ch % 2 == 0)
        def _():
            _drain(ch, sb0, db0, wb0, sem0)
            _process(sb0, db0, wb0)

        @pl.when(ch % 2 == 1)
        def _():
            _drain(ch, sb1, db1, wb1, sem1)
            _process(sb1, db1, wb1)
        return 0

    lax.fori_loop(0, NC2, _loop, 0)

    pltpu.sync_copy(ubuf, u_out.at[cid, pl.ds(sid * 8, 8)])


def _phase2(ht, srcm, dstm, w):
    mesh = plsc.VectorSubcoreMesh(
        core_axis_name="c", subcore_axis_name="s", num_cores=2,
        num_subcores=16)
    f = pl.kernel(
        _p2_body,
        out_type=jax.ShapeDtypeStruct((2, D, NP2), jnp.float32),
        mesh=mesh,
        compiler_params=pltpu.CompilerParams(needs_layout_passes=False),
        scratch_types=[
            pltpu.VMEM((4, NP2), jnp.int32),
            pltpu.VMEM((8, NP2), jnp.float32),
            pltpu.VMEM((EC // 128, 128), jnp.int32),
            pltpu.VMEM((EC // 128, 128), jnp.int32),
            pltpu.VMEM((EC,), jnp.float32),
            pltpu.VMEM((EC // 128, 128), jnp.int32),
            pltpu.VMEM((EC // 128, 128), jnp.int32),
            pltpu.VMEM((EC,), jnp.float32),
            pltpu.SemaphoreType.DMA,
            pltpu.SemaphoreType.DMA,
        ],
    )
    return f(ht, srcm, dstm, w)


def _head_body(u_ref, den_ref, bg_ref, wih_ref, whh_ref, bl_ref,
               w1_ref, b1_ref, w2_ref, b2_ref, out_ref):
    u = u_ref[0] + u_ref[1]
    den = den_ref[0] + den_ref[1]
    agg = u / (den + 1e-9)
    x = agg + bg_ref[...]
    nodes = jnp.where(x > 0.0, x, jnp.exp(x) - 1.0)
    cols = lax.broadcasted_iota(jnp.int32, (1, NP2), 1)
    valid = cols < N
    nodes = jnp.where(valid, nodes, 0.0)

    q_star = jnp.zeros((1, 2 * D), jnp.float32)
    hh = jnp.zeros((1, D), jnp.float32)
    cc = jnp.zeros((1, D), jnp.float32)
    for _ in range(3):
        z = q_star @ wih_ref[...] + hh @ whh_ref[...] + bl_ref[...]
        zi = z[:, 0:D]
        zf = z[:, D:2 * D]
        zg = z[:, 2 * D:3 * D]
        zo = z[:, 3 * D:4 * D]
        cc = jax.nn.sigmoid(zf) * cc + jax.nn.sigmoid(zi) * jnp.tanh(zg)
        hh = jax.nn.sigmoid(zo) * jnp.tanh(cc)
        logits = hh @ nodes
        logits = jnp.where(valid, logits, -3.4e38)
        mx = jnp.max(logits)
        aw = jnp.exp(logits - mx)
        aw = aw / jnp.sum(aw)
        r = lax.dot_general(aw, nodes, (((1,), (1,)), ((), ())))
        q_star = jnp.concatenate([hh, r], axis=1)

    xm = jnp.maximum(q_star @ w1_ref[...] + b1_ref[...], 0.0)
    out_ref[...] = xm @ w2_ref[...] + b2_ref[...]


def _head(u2, den2, bg, wih, whh, bl, w1, b1, w2, b2):
    return pl.pallas_call(
        _head_body,
        out_shape=jax.ShapeDtypeStruct((1, 1), jnp.float32),
    )(u2, den2, bg, wih, whh, bl, w1, b1, w2, b2)


def kernel(node_feats, edge_index, gh, Wn, Wg, a_src, a_dst, b_gat,
           W_ih, W_hh, b_lstm, W1, b1, W2, b2):
    nf_pad = jnp.pad(node_feats, ((0, NP - N), (0, 0)))
    ht, ss, sd, mout = _project(
        nf_pad, Wn, gh, Wg, a_src.reshape(D, 1), a_dst.reshape(D, 1))

    pad = EP - E
    srcm = jnp.concatenate(
        [edge_index[0], jnp.zeros((pad,), jnp.int32)]).reshape(EROWS, C)
    dstm = jnp.concatenate(
        [edge_index[1], jnp.full((pad,), N, jnp.int32)]).reshape(EROWS, C)
    mval = jnp.broadcast_to(jnp.reshape(mout, ()), (16,))

    w, den2 = _phase1(srcm, dstm, ss.reshape(NP), sd.reshape(NP), mval)
    u2 = _phase2(ht, srcm.reshape(EP // 128, 128),
                 dstm.reshape(EP // 128, 128), w)

    return _head(
        u2, den2[:, :NP2].reshape(2, 1, NP2), b_gat.reshape(D, 1), W_ih, W_hh,
        b_lstm.reshape(1, 4 * D), W1, b1.reshape(1, D), W2,
        b2.reshape(1, 1))


# speedup vs baseline: 1.6081x; 1.6081x over previous
import functools

import jax
import jax.numpy as jnp
from jax import lax
from jax.experimental import pallas as pl
from jax.experimental.pallas import tpu as pltpu
from jax.experimental.pallas import tpu_sc as plsc

N = 10000
D = 128
NP = 10240
TR = NP // 16
E = 320000
C = 64
GRP = 16
NW = 32
CW = ((-(-E // (NW * C)) + GRP - 1) // GRP) * GRP
NG = CW // GRP
EP = NW * CW * C
EROWS = EP // C
EC = 1024
NC2 = (EP // 2) // EC
HC = 64
NP2 = 10112
BLK = 256


def _proj_body(nf, wn, gh, wg, asrc, adst, ht_out, ss_out, sd_out, m_out,
               macc):
    i = pl.program_id(0)
    hg = gh[...] @ wg[...]
    hb = nf[...] @ wn[...] + hg
    ht_out[...] = jnp.transpose(hb)
    sb = hb @ asrc[...]
    db = hb @ adst[...]
    ss_out[...] = sb
    sd_out[...] = db

    @pl.when(i == 0)
    def _():
        macc[0] = jnp.float32(-3.4e38)
        macc[1] = jnp.float32(-3.4e38)

    macc[0] = jnp.maximum(macc[0], jnp.max(sb))
    macc[1] = jnp.maximum(macc[1], jnp.max(db))

    @pl.when(i == pl.num_programs(0) - 1)
    def _():
        m_out[...] = jnp.full((1, 1), jnp.maximum(macc[0] + macc[1], 0.0))


def _project(nf_pad, Wn, gh, Wg, asrc, adst):
    grid = NP // BLK
    return pl.pallas_call(
        _proj_body,
        grid=(grid,),
        in_specs=[
            pl.BlockSpec((BLK, D), lambda i: (i, 0)),
            pl.BlockSpec((D, D), lambda i: (0, 0)),
            pl.BlockSpec((1, D), lambda i: (0, 0)),
            pl.BlockSpec((D, D), lambda i: (0, 0)),
            pl.BlockSpec((D, 1), lambda i: (0, 0)),
            pl.BlockSpec((D, 1), lambda i: (0, 0)),
        ],
        out_specs=[
            pl.BlockSpec((D, BLK), lambda i: (0, i)),
            pl.BlockSpec((BLK, 1), lambda i: (i, 0)),
            pl.BlockSpec((BLK, 1), lambda i: (i, 0)),
            pl.BlockSpec((1, 1), lambda i: (0, 0)),
        ],
        out_shape=[
            jax.ShapeDtypeStruct((D, NP), jnp.float32),
            jax.ShapeDtypeStruct((NP, 1), jnp.float32),
            jax.ShapeDtypeStruct((NP, 1), jnp.float32),
            jax.ShapeDtypeStruct((1, 1), jnp.float32),
        ],
        scratch_shapes=[pltpu.SMEM((2,), jnp.float32)],
    )(nf_pad, Wn, gh, Wg, asrc, adst)


def _p1_body(srcm_hbm, dstm_hbm, ssrc_hbm, sdst_hbm, mval_hbm,
             w_out, den_out,
             ssrc_v, sdst_v, sg_src, sg_dst, wb, mv_v, zden, den_sh):
    cid = lax.axis_index("c")
    sid = lax.axis_index("s")
    wid = sid * 2 + cid

    pltpu.sync_copy(ssrc_hbm, ssrc_v)
    pltpu.sync_copy(sdst_hbm, sdst_v)
    pltpu.sync_copy(mval_hbm, mv_v)

    def _zden(r, _):
        zden[pl.ds(r * 16, 16)] = jnp.zeros((16,), jnp.float32)
        return 0
    lax.fori_loop(0, TR // 16, _zden, 0)
    pltpu.sync_copy(zden, den_sh.at[pl.ds(sid * TR, TR)])
    plsc.subcore_barrier()

    mvec = mv_v[...]

    def _group(g, _):
        base = wid * CW + g * GRP
        pltpu.sync_copy(srcm_hbm.at[pl.ds(base, GRP)], sg_src)
        pltpu.sync_copy(dstm_hbm.at[pl.ds(base, GRP)], sg_dst)

        def _chunk(j, _):
            for k in range(C // 16):
                si = sg_src[j, pl.ds(k * 16, 16)]
                di = sg_dst[j, pl.ds(k * 16, 16)]
                a = plsc.load_gather(ssrc_v, [si])
                b = plsc.load_gather(sdst_v, [di])
                pre = a + b
                e = jnp.where(pre >= 0.0, pre, 0.2 * pre)
                wb[pl.ds(k * 16, 16)] = jnp.exp(e - mvec)
            pltpu.sync_copy(wb, w_out.at[pl.ds((base + j) * C, C)])
            pltpu.sync_copy(wb, den_sh.at[sg_dst.at[j]], add=True)
            return 0

        lax.fori_loop(0, GRP, _chunk, 0)
        return 0

    lax.fori_loop(0, NG, _group, 0)
    plsc.subcore_barrier()
    pltpu.sync_copy(den_sh.at[pl.ds(sid * TR, TR)],
                    den_out.at[cid, pl.ds(sid * TR, TR)])


def _phase1(srcm, dstm, ssrc, sdst, mval):
    mesh = plsc.VectorSubcoreMesh(
        core_axis_name="c", subcore_axis_name="s", num_cores=2,
        num_subcores=16)
    f = pl.kernel(
        _p1_body,
        out_type=[
            jax.ShapeDtypeStruct((EP,), jnp.float32),
            jax.ShapeDtypeStruct((2, NP), jnp.float32),
        ],
        mesh=mesh,
        compiler_params=pltpu.CompilerParams(needs_layout_passes=False),
        scratch_types=[
            pltpu.VMEM((NP,), jnp.float32),
            pltpu.VMEM((NP,), jnp.float32),
            pltpu.VMEM((GRP, C), jnp.int32),
            pltpu.VMEM((GRP, C), jnp.int32),
            pltpu.VMEM((C,), jnp.float32),
            pltpu.VMEM((16,), jnp.float32),
            pltpu.VMEM((TR,), jnp.float32),
            pltpu.VMEM_SHARED((NP,), jnp.float32),
        ],
    )
    return f(srcm, dstm, ssrc, sdst, mval)


def _p2_body(ht_hbm, srcm_hbm, dstm_hbm, w_hbm, u_out,
             table, ubuf, sb0, db0, wb0, sb1, db1, wb1,
             sem0, sem1):
    cid = lax.axis_index("c")
    sid = lax.axis_index("s")

    pltpu.sync_copy(ht_hbm.at[pl.ds(sid * 8, 8), pl.ds(0, NP2)], ubuf)

    def _packcols(i, _):
        for j in range(4):
            a = ubuf[2 * j, pl.ds(i * 16, 16)]
            b = ubuf[2 * j + 1, pl.ds(i * 16, 16)]
            pk = plsc.pack(a, b, format=plsc.PackFormat.INTERLEAVED)
            table[j, pl.ds(i * 16, 16)] = plsc.bitcast(pk, jnp.int32)
        return 0
    lax.fori_loop(0, NP2 // 16, _packcols, 0)

    def _zu(r, _):
        for j in range(8):
            ubuf[j, pl.ds(r * 16, 16)] = jnp.zeros((16,), jnp.float32)
        return 0
    lax.fori_loop(0, NP2 // 16, _zu, 0)

    jconst = [jnp.full((16,), j, jnp.int32) for j in range(8)]
    ebase0 = cid * (EP // 2)

    def _fire(ch, sb, db, wb, sem):
        ebase = ebase0 + ch * EC
        pltpu.async_copy(srcm_hbm.at[pl.ds(ebase, EC)], sb, sem)
        pltpu.async_copy(dstm_hbm.at[pl.ds(ebase, EC)], db, sem)
        pltpu.async_copy(w_hbm.at[pl.ds(ebase, EC)], wb, sem)

    def _drain(ch, sb, db, wb, sem):
        ebase = ebase0 + ch * EC
        pltpu.make_async_copy(
            srcm_hbm.at[pl.ds(ebase, EC)], sb, sem).wait()
        pltpu.make_async_copy(
            dstm_hbm.at[pl.ds(ebase, EC)], db, sem).wait()
        pltpu.make_async_copy(
            w_hbm.at[pl.ds(ebase, EC)], wb, sem).wait()

    def _process(sb, db, wb):
        @plsc.parallel_loop(0, EC // 16, 1, unroll=2)
        def _step(i):
            off = i * 16
            srcv = sb[pl.ds(off, 16)]
            dstv = db[pl.ds(off, 16)]
            wv = wb[pl.ds(off, 16)]
            for j in range(4):
                g = plsc.load_gather(table, [jconst[j], srcv])
                bf = plsc.bitcast(g, jnp.bfloat16)
                fa, fb = plsc.unpack(
                    bf, format=plsc.PackFormat.INTERLEAVED)
                plsc.addupdate_scatter(
                    ubuf, [jconst[2 * j], dstv], wv * fa)
                plsc.addupdate_scatter(
                    ubuf, [jconst[2 * j + 1], dstv], wv * fb)

    _fire(0, sb0, db0, wb0, sem0)

    def _loop(ch, _):
        @pl.when(jnp.logical_and(ch + 1 < NC2, ch % 2 == 0))
        def _():
            _fire(ch + 1, sb1, db1, wb1, sem1)

        @pl.when(jnp.logical_and(ch + 1 < NC2, ch % 2 == 1))
        def _():
            _fire(ch + 1, sb0, db0, wb0, sem0)

        @pl.when(ch % 2 == 0)
        def _():
            _drain(ch, sb0, db0, wb0, sem0)
            _process(sb0, db0, wb0)

        @pl.when(ch % 2 == 1)
        def _():
            _drain(ch, sb1, db1, wb1, sem1)
            _process(sb1, db1, wb1)
        return 0

    lax.fori_loop(0, NC2, _loop, 0)

    pltpu.sync_copy(ubuf, u_out.at[cid, pl.ds(sid * 8, 8)])


def _phase2(ht, srcm, dstm, w):
    mesh = plsc.VectorSubcoreMesh(
        core_axis_name="c", subcore_axis_name="s", num_cores=2,
        num_subcores=16)
    f = pl.kernel(
        _p2_body,
        out_type=jax.ShapeDtypeStruct((2, D, NP2), jnp.float32),
        mesh=mesh,
        compiler_params=pltpu.CompilerParams(needs_layout_passes=False),
        scratch_types=[
            pltpu.VMEM((4, NP2), jnp.int32),
            pltpu.VMEM((8, NP2), jnp.float32),
            pltpu.VMEM((EC,), jnp.int32),
            pltpu.VMEM((EC,), jnp.int32),
            pltpu.VMEM((EC,), jnp.float32),
            pltpu.VMEM((EC,), jnp.int32),
            pltpu.VMEM((EC,), jnp.int32),
            pltpu.VMEM((EC,), jnp.float32),
            pltpu.SemaphoreType.DMA,
            pltpu.SemaphoreType.DMA,
        ],
    )
    return f(ht, srcm, dstm, w)


def _head_body(u_ref, den_ref, bg_ref, wih_ref, whh_ref, bl_ref,
               w1_ref, b1_ref, w2_ref, b2_ref, out_ref):
    u = u_ref[0] + u_ref[1]
    den = den_ref[0] + den_ref[1]
    agg = u / (den + 1e-9)
    x = agg + bg_ref[...]
    nodes = jnp.where(x > 0.0, x, jnp.exp(x) - 1.0)
    cols = lax.broadcasted_iota(jnp.int32, (1, NP2), 1)
    valid = cols < N
    nodes = jnp.where(valid, nodes, 0.0)

    q_star = jnp.zeros((1, 2 * D), jnp.float32)
    hh = jnp.zeros((1, D), jnp.float32)
    cc = jnp.zeros((1, D), jnp.float32)
    for _ in range(3):
        z = q_star @ wih_ref[...] + hh @ whh_ref[...] + bl_ref[...]
        zi = z[:, 0:D]
        zf = z[:, D:2 * D]
        zg = z[:, 2 * D:3 * D]
        zo = z[:, 3 * D:4 * D]
        cc = jax.nn.sigmoid(zf) * cc + jax.nn.sigmoid(zi) * jnp.tanh(zg)
        hh = jax.nn.sigmoid(zo) * jnp.tanh(cc)
        logits = hh @ nodes
        logits = jnp.where(valid, logits, -3.4e38)
        mx = jnp.max(logits)
        aw = jnp.exp(logits - mx)
        aw = aw / jnp.sum(aw)
        r = lax.dot_general(aw, nodes, (((1,), (1,)), ((), ())))
        q_star = jnp.concatenate([hh, r], axis=1)

    xm = jnp.maximum(q_star @ w1_ref[...] + b1_ref[...], 0.0)
    out_ref[...] = xm @ w2_ref[...] + b2_ref[...]


def _head(u2, den2, bg, wih, whh, bl, w1, b1, w2, b2):
    return pl.pallas_call(
        _head_body,
        out_shape=jax.ShapeDtypeStruct((1, 1), jnp.float32),
    )(u2, den2, bg, wih, whh, bl, w1, b1, w2, b2)


def kernel(node_feats, edge_index, gh, Wn, Wg, a_src, a_dst, b_gat,
           W_ih, W_hh, b_lstm, W1, b1, W2, b2):
    nf_pad = jnp.pad(node_feats, ((0, NP - N), (0, 0)))
    ht, ss, sd, mout = _project(
        nf_pad, Wn, gh, Wg, a_src.reshape(D, 1), a_dst.reshape(D, 1))

    pad = EP - E
    srcm = jnp.concatenate(
        [edge_index[0], jnp.zeros((pad,), jnp.int32)]).reshape(EROWS, C)
    dstm = jnp.concatenate(
        [edge_index[1], jnp.full((pad,), N, jnp.int32)]).reshape(EROWS, C)
    mval = jnp.broadcast_to(jnp.reshape(mout, ()), (16,))

    w, den2 = _phase1(srcm, dstm, ss.reshape(NP), sd.reshape(NP), mval)
    u2 = _phase2(ht, srcm.reshape(EP), dstm.reshape(EP), w)

    return _head(
        u2, den2[:, :NP2].reshape(2, 1, NP2), b_gat.reshape(D, 1), W_ih, W_hh,
        b_lstm.reshape(1, 4 * D), W1, b1.reshape(1, D), W2,
        b2.reshape(1, 1))
